# baseline (speedup 1.0000x reference)
import functools

import jax
import jax.numpy as jnp
from jax import lax
from jax.experimental import pallas as pl
from jax.experimental.pallas import tpu as pltpu
from jax.experimental.pallas import tpu_sc as plsc

VOCAB = 1000000
EMB = 32
B = 16384
L = 20
N = B * L

NUM_CORES = 2
NUM_SUBCORES = 16
NW = NUM_CORES * NUM_SUBCORES

E_PER_W = 31248
ECHUNK = 496
NECH = E_PER_W // ECHUNK
NEG = ECHUNK // 16
E_TAIL_BASE = NW * E_PER_W

B_PER_W = B // NW
ROWS_PER_W = B_PER_W * L
CHUNK_B = 16
CHUNK = CHUNK_B * L
N_CHUNKS = B_PER_W // CHUNK_B
NGB = CHUNK_B // 16


def _make_gather():
    mesh = plsc.VectorSubcoreMesh(core_axis_name="c", subcore_axis_name="s")

    @functools.partial(
        pl.kernel,
        mesh=mesh,
        out_type=jax.ShapeDtypeStruct((L, EMB, B), jnp.float32),
        scratch_types=[
            pltpu.HBM((VOCAB, EMB), jnp.float32),
            pltpu.VMEM((2, EMB, ECHUNK), jnp.float32),
            pltpu.VMEM((2, ECHUNK, EMB), jnp.float32),
            pltpu.VMEM((ROWS_PER_W,), jnp.int32),
            pltpu.VMEM((2, CHUNK, EMB), jnp.float32),
            pltpu.VMEM((2, L, EMB, CHUNK_B), jnp.float32),
            pltpu.SemaphoreType.DMA((2,)),
            pltpu.SemaphoreType.DMA((2,)),
            pltpu.SemaphoreType.DMA((2,)),
            pltpu.SemaphoreType.DMA((2,)),
            pltpu.SemaphoreType.REGULAR,
        ],
        compiler_params=pltpu.CompilerParams(
            use_tc_tiling_on_sc=False, needs_layout_passes=False
        ),
    )
    def gather_kernel(
        idx_hbm,
        tableT_hbm,
        out_hbm,
        tbl_lin,
        slab_v,
        wrows_v,
        idx_v,
        rows_v,
        t_v,
        ssem,
        esem,
        gsem,
        wsem,
        bsem,
    ):
        wid = lax.axis_index("s") * NUM_CORES + lax.axis_index("c")
        lanes = lax.iota(jnp.int32, 16)

        e_base = wid * E_PER_W

        def slab_args(j, pb):
            return (
                tableT_hbm.at[:, pl.ds(e_base + j * ECHUNK, ECHUNK)],
                slab_v.at[pb],
                ssem.at[pb],
            )

        def wrow_args(j, pb):
            return (
                wrows_v.at[pb],
                tbl_lin.at[pl.ds(e_base + j * ECHUNK, ECHUNK)],
                esem.at[pb],
            )

        def transpose_slab(pb):
            @pl.loop(0, NEG)
            def _eg(eg):
                e0 = eg * 16
                rvec = lanes + e0
                for cb in range(0, EMB, 8):
                    xs = [
                        slab_v[pb, cb + i, pl.ds(e0, 16)] for i in range(8)
                    ]
                    for i in range(8):
                        plsc.store_scatter(
                            wrows_v.at[pb],
                            [rvec, jnp.full((16,), cb + i, jnp.int32)],
                            xs[i],
                        )

        def do_chunk(j, pb, last):
            pltpu.make_async_copy(*slab_args(j, pb)).wait()
            if not last:
                pltpu.async_copy(*slab_args(j + 1, 1 - pb))
            transpose_slab(pb)
            pltpu.async_copy(*wrow_args(j, pb))

        @pl.when(wid < 2)
        def _tail_rows():
            e_t = E_TAIL_BASE + wid * 32
            pltpu.sync_copy(
                tableT_hbm.at[:, pl.ds(e_t, 32)], slab_v.at[0, :, pl.ds(0, 32)]
            )
            for eg in range(2):
                rvec = lanes + eg * 16
                for c in range(EMB):
                    x = slab_v[0, c, pl.ds(eg * 16, 16)]
                    plsc.store_scatter(
                        wrows_v.at[0],
                        [rvec, jnp.full((16,), c, jnp.int32)],
                        x,
                    )
            pltpu.sync_copy(
                wrows_v.at[0, pl.ds(0, 32)], tbl_lin.at[pl.ds(e_t, 32)]
            )

        pltpu.async_copy(*slab_args(0, 0))

        @pl.loop(0, NECH - 1, step=2)
        def _echunks(j0):
            for pb in range(2):
                j = j0 + pb

                @pl.when(j >= 2)
                def _wrow_drain():
                    pltpu.make_async_copy(*wrow_args(j - 2, pb)).wait()

                do_chunk(j, pb, False)

        jf = NECH - 1
        pf = jf % 2
        pltpu.make_async_copy(*wrow_args(jf - 2, pf)).wait()
        do_chunk(jf, pf, True)
        pltpu.make_async_copy(*wrow_args(jf - 1, 1 - pf)).wait()
        pltpu.make_async_copy(*wrow_args(jf, pf)).wait()

        plsc.subcore_barrier()
        pltpu.core_barrier(bsem, core_axis_name="c")
        plsc.subcore_barrier()

        base = wid * ROWS_PER_W
        b_base = wid * B_PER_W
        pltpu.sync_copy(idx_hbm.at[pl.ds(base, ROWS_PER_W)], idx_v)

        def gather_args(j, p):
            return (
                tbl_lin.at[idx_v.at[pl.ds(j * CHUNK, CHUNK)]],
                rows_v.at[p],
                gsem.at[p],
            )

        pltpu.async_copy(*gather_args(0, 0))

        @pl.loop(0, N_CHUNKS, step=2)
        def _chunks(j0):
            for p in range(2):
                j = j0 + p
                b0 = b_base + j * CHUNK_B
                pltpu.make_async_copy(*gather_args(j, p)).wait()

                @pl.when(j + 1 < N_CHUNKS)
                def _next():
                    pltpu.async_copy(*gather_args(j + 1, (p + 1) % 2))

                @pl.when(j >= 2)
                def _drains():
                    @pl.loop(0, L)
                    def _drain(l):
                        pltpu.make_async_copy(
                            t_v.at[p, l],
                            out_hbm.at[
                                l, :, pl.ds(b_base + (j - 2) * CHUNK_B, CHUNK_B)
                            ],
                            wsem.at[p],
                        ).wait()

                @pl.loop(0, L)
                def _transpose(l):
                    for g in range(NGB):
                        rvec = (lanes + g * 16) * L + l
                        for cb in range(0, EMB, 8):
                            xs = [
                                plsc.load_gather(
                                    rows_v.at[p],
                                    [rvec, jnp.full((16,), cb + i, jnp.int32)],
                                )
                                for i in range(8)
                            ]
                            for i in range(8):
                                t_v[p, l, cb + i, pl.ds(g * 16, 16)] = xs[i]

                @pl.loop(0, L)
                def _writeback(l):
                    pltpu.async_copy(
                        t_v.at[p, l],
                        out_hbm.at[l, :, pl.ds(b0, CHUNK_B)],
                        wsem.at[p],
                    )

        for j in range(N_CHUNKS - 2, N_CHUNKS):
            p = j % 2

            @pl.loop(0, L)
            def _drain_tail(l):
                pltpu.make_async_copy(
                    t_v.at[p, l],
                    out_hbm.at[l, :, pl.ds(b_base + j * CHUNK_B, CHUNK_B)],
                    wsem.at[p],
                ).wait()

    return gather_kernel


_gather = _make_gather()


@jax.jit
def kernel(indices, table):
    flat_idx = indices.reshape(N)
    out_lcb = _gather(flat_idx, table.T)
    return out_lcb.transpose(2, 0, 1)


# speedup vs baseline: 4.9751x; 4.9751x over previous
import functools

import jax
import jax.numpy as jnp
from jax import lax
from jax.experimental import pallas as pl
from jax.experimental.pallas import tpu as pltpu
from jax.experimental.pallas import tpu_sc as plsc

VOCAB = 1000000
EMB = 32
B = 16384
L = 20
N = B * L

NUM_CORES = 2
NUM_SUBCORES = 16
NW = NUM_CORES * NUM_SUBCORES
B_PER_W = B // NW
ROWS_PER_W = B_PER_W * L
CHUNK_B = 16
CHUNK = CHUNK_B * L
N_CHUNKS = B_PER_W // CHUNK_B


def _make_gather():
    mesh = plsc.VectorSubcoreMesh(core_axis_name="c", subcore_axis_name="s")

    @functools.partial(
        pl.kernel,
        mesh=mesh,
        out_type=jax.ShapeDtypeStruct((L, EMB, B), jnp.float32),
        scratch_types=[
            pltpu.VMEM((ROWS_PER_W,), jnp.int32),
            pltpu.VMEM((2, CHUNK, EMB), jnp.float32),
            pltpu.VMEM((2, L, EMB, CHUNK_B), jnp.float32),
            pltpu.SemaphoreType.DMA((2,)),
            pltpu.SemaphoreType.DMA((2,)),
        ],
        compiler_params=pltpu.CompilerParams(
            use_tc_tiling_on_sc=False, needs_layout_passes=False
        ),
    )
    def gather_kernel(idx_hbm, table_hbm, out_hbm, idx_v, rows_v, t_v, gsem, wsem):
        wid = lax.axis_index("s") * NUM_CORES + lax.axis_index("c")
        base = wid * ROWS_PER_W
        b_base = wid * B_PER_W
        pltpu.sync_copy(idx_hbm.at[pl.ds(base, ROWS_PER_W)], idx_v)

        lanes = lax.iota(jnp.int32, 16)

        def gather_args(j, p):
            return (
                table_hbm.at[idx_v.at[pl.ds(j * CHUNK, CHUNK)]],
                rows_v.at[p],
                gsem.at[p],
            )

        pltpu.async_copy(*gather_args(0, 0))

        @pl.loop(0, N_CHUNKS, step=2)
        def _chunks(j0):
            for p in range(2):
                j = j0 + p
                b0 = b_base + j * CHUNK_B
                pltpu.make_async_copy(*gather_args(j, p)).wait()

                @pl.when(j + 1 < N_CHUNKS)
                def _next():
                    pltpu.async_copy(*gather_args(j + 1, (p + 1) % 2))

                @pl.when(j >= 2)
                def _drains():
                    @pl.loop(0, L)
                    def _drain(l):
                        pltpu.make_async_copy(
                            t_v.at[p, l],
                            out_hbm.at[
                                l, :, pl.ds(b_base + (j - 2) * CHUNK_B, CHUNK_B)
                            ],
                            wsem.at[p],
                        ).wait()

                for l in range(L):
                    for bb in range(0, CHUNK_B, 4):
                        xs = []
                        for b in range(bb, bb + 4):
                            row = b * L + l
                            xs.append(rows_v[p, row, pl.ds(0, 16)])
                            xs.append(rows_v[p, row, pl.ds(16, 16)])
                        for k in range(4):
                            b = bb + k
                            bvec = jnp.full((16,), b, jnp.int32)
                            plsc.store_scatter(
                                t_v.at[p, l], [lanes, bvec], xs[2 * k]
                            )
                            plsc.store_scatter(
                                t_v.at[p, l], [lanes + 16, bvec], xs[2 * k + 1]
                            )

                @pl.loop(0, L)
                def _writeback(l):
                    pltpu.async_copy(
                        t_v.at[p, l],
                        out_hbm.at[l, :, pl.ds(b0, CHUNK_B)],
                        wsem.at[p],
                    )

        for j in range(N_CHUNKS - 2, N_CHUNKS):
            p = j % 2

            @pl.loop(0, L)
            def _drain_tail(l):
                pltpu.make_async_copy(
                    t_v.at[p, l],
                    out_hbm.at[l, :, pl.ds(b_base + j * CHUNK_B, CHUNK_B)],
                    wsem.at[p],
                ).wait()

    return gather_kernel


_gather = _make_gather()


@jax.jit
def kernel(indices, table):
    flat_idx = indices.reshape(N)
    out_lcb = _gather(flat_idx, table)
    return out_lcb.transpose(2, 0, 1)
